# baseline (speedup 1.0000x reference)
import functools

import jax
import jax.numpy as jnp
from jax import lax
from jax.experimental import pallas as pl
from jax.experimental.pallas import tpu as pltpu
from jax.experimental.pallas import tpu_sc as plsc

E = 160000
N = 10000
D = 256
BE = 3200
NBLK = E // BE

NW = 32
NPAD = 10240
NT = NPAD // NW
K = 128
EK = E - K


def _mlp_kernel(xi_ref, xj_ref, ea_ref, w_ref, b_ref, m_ref):
    xi = xi_ref[...].astype(jnp.bfloat16)
    xj = xj_ref[...].astype(jnp.bfloat16)
    ea = ea_ref[...].astype(jnp.bfloat16)
    w = w_ref[...].astype(jnp.bfloat16)
    acc = jax.lax.dot_general(xi, w[0:D, :], (((1,), (0,)), ((), ())),
                              preferred_element_type=jnp.float32)
    acc += jax.lax.dot_general(xj, w[D:2 * D, :], (((1,), (0,)), ((), ())),
                               preferred_element_type=jnp.float32)
    acc += jax.lax.dot_general(ea, w[2 * D:3 * D, :], (((1,), (0,)), ((), ())),
                               preferred_element_type=jnp.float32)
    m_ref[...] = jnp.maximum(acc + b_ref[...], 0.0)


def _segsum_body(m_hbm, rec_hbm, bounds_hbm, aggr_hbm,
                 acc, stage, idsv, bv, dma_sem):
    nc = jax.lax.axis_size("c")
    wid = lax.axis_index("s") * nc + lax.axis_index("c")
    base = wid * NT

    pltpu.sync_copy(bounds_hbm, bv)
    vb = bv[pl.ds(2 * wid, 16)]
    sw = vb[0]
    ew = vb[1]
    s_al = (sw // 8) * 8
    nch = lax.div(ew - s_al + (K - 1), K)

    zv = jnp.zeros((16,), jnp.float32)

    def zero_body(r, carry):
        for k in range(16):
            acc[r, pl.ds(k * 16, 16)] = zv
        return carry

    lax.fori_loop(0, NT + 1, zero_body, 0)

    def chunk_body(c, carry):
        start_u = s_al + c * K
        sc = jnp.minimum(start_u, EK)
        pltpu.sync_copy(m_hbm.at[pl.ds(sc, K)], stage)
        pltpu.sync_copy(rec_hbm.at[pl.ds(sc, K)], idsv)
        lo = jnp.maximum(sw, start_u)

        def group_body(g, gcarry):
            idg = idsv[pl.ds(g * 16, 16)]
            gidx = sc + g * 16 + lax.iota(jnp.int32, 16)
            valid = (gidx >= lo) & (gidx < ew)
            rel = jnp.where(valid, idg - base, NT)
            for e in range(16):
                r = rel[e]
                row = g * 16 + e
                for k in range(16):
                    x = stage[row, pl.ds(k * 16, 16)]
                    plsc.addupdate(acc.at[r, pl.ds(k * 16, 16)], x)
            return gcarry

        lax.fori_loop(0, K // 16, group_body, 0)
        return carry

    lax.fori_loop(0, nch, chunk_body, 0)
    pltpu.sync_copy(acc.at[pl.ds(0, NT)], aggr_hbm.at[pl.ds(base, NT)])


@jax.jit
def _run(x_i, x_j, recipients, edge_attr, W, b):
    m = pl.pallas_call(
        _mlp_kernel,
        grid=(NBLK,),
        in_specs=[
            pl.BlockSpec((BE, D), lambda i: (i, 0)),
            pl.BlockSpec((BE, D), lambda i: (i, 0)),
            pl.BlockSpec((BE, D), lambda i: (i, 0)),
            pl.BlockSpec((3 * D, D), lambda i: (0, 0)),
            pl.BlockSpec((1, D), lambda i: (0, 0)),
        ],
        out_specs=pl.BlockSpec((BE, D), lambda i: (i, 0)),
        out_shape=jax.ShapeDtypeStruct((E, D), jnp.float32),
    )(x_i, x_j, edge_attr, W, b.reshape(1, D))

    node_edges = jnp.searchsorted(
        recipients, jnp.arange(0, NPAD + NT, NT, dtype=jnp.int32)
    ).astype(jnp.int32)
    bounds = jnp.stack(
        [node_edges[:-1], node_edges[1:]], axis=1).reshape(2 * NW)
    bounds = jnp.pad(bounds, (0, 16))

    mesh = plsc.VectorSubcoreMesh(core_axis_name="c", subcore_axis_name="s")
    segsum = pl.kernel(
        _segsum_body,
        out_type=jax.ShapeDtypeStruct((NPAD, D), jnp.float32),
        mesh=mesh,
        scratch_types=[
            pltpu.VMEM((NT + 1, D), jnp.float32),
            pltpu.VMEM((K, D), jnp.float32),
            pltpu.VMEM((K,), jnp.int32),
            pltpu.VMEM((2 * NW + 16,), jnp.int32),
            pltpu.SemaphoreType.DMA,
        ],
    )
    aggr = segsum(m, recipients, bounds)
    return aggr[:N], m


def kernel(x_i, x_j, recipients, edge_attr, num_segments, W, b):
    aggr, m = _run(x_i, x_j, recipients, edge_attr, W, b)
    return (aggr, m)
